# Initial kernel scaffold; baseline (speedup 1.0000x reference)
import jax
import jax.numpy as jnp
from jax.experimental import pallas as pl


def kernel(x, edge_index, W_rel, b_rel, W_root):
    raise NotImplementedError("write your pallas kernel here")


# speedup vs baseline: 5.3481x; 5.3481x over previous
import functools

import jax
import jax.numpy as jnp
from jax import lax
from jax.experimental import pallas as pl
from jax.experimental.pallas import tpu as pltpu
from jax.experimental.pallas import tpu_sc as plsc

N_NODES = 10000
N_EDGES = 320000
D = 128

NC = 2
NS = 16
NW = NC * NS
EPW = N_EDGES // NW
K = 80
NCHUNK = EPW // K
N_PAD = 10240
ROWS_PER_TILE = N_PAD // NS


def _sc_segment_sum(x, src, dst, zeros):
    mesh = plsc.VectorSubcoreMesh(core_axis_name="c", subcore_axis_name="s")

    @functools.partial(
        pl.kernel,
        out_type=jax.ShapeDtypeStruct((NC, N_PAD, D), jnp.float32),
        mesh=mesh,
        scratch_types=[
            pltpu.VMEM((K,), jnp.int32),
            pltpu.VMEM((K,), jnp.int32),
            pltpu.VMEM((K, D), jnp.float32),
            pltpu.VMEM_SHARED((N_PAD, D), jnp.float32),
            pltpu.SemaphoreType.DMA,
        ],
    )
    def seg_sum(x_hbm, src_hbm, dst_hbm, zeros_hbm, out_hbm,
                src_v, dst_v, rows_v, acc, sem):
        cid = lax.axis_index("c")
        sid = lax.axis_index("s")
        wid = sid * NC + cid

        zbase = sid * ROWS_PER_TILE
        pltpu.sync_copy(zeros_hbm.at[pl.ds(zbase, ROWS_PER_TILE)],
                        acc.at[pl.ds(zbase, ROWS_PER_TILE)])
        plsc.subcore_barrier()

        ebase = wid * EPW

        def body(i, _):
            base = ebase + i * K
            pltpu.sync_copy(src_hbm.at[pl.ds(base, K)], src_v)
            pltpu.sync_copy(dst_hbm.at[pl.ds(base, K)], dst_v)
            pltpu.async_copy(x_hbm.at[src_v], rows_v, sem).wait()
            pltpu.sync_copy(rows_v, acc.at[dst_v], add=True)
            return 0

        lax.fori_loop(0, NCHUNK, body, 0)
        plsc.subcore_barrier()

        pltpu.sync_copy(acc.at[pl.ds(zbase, ROWS_PER_TILE)],
                        out_hbm.at[cid, pl.ds(zbase, ROWS_PER_TILE)])

    return seg_sum(x, src, dst, zeros)


BLK = 1000


def _tc_combine(p0, p1, x, wr_t, wt_t, b):
    def body(p0_ref, p1_ref, x_ref, wr_ref, wt_ref, b_ref, o_ref):
        agg = p0_ref[...] + p1_ref[...]
        o_ref[...] = (
            jnp.dot(agg, wr_ref[...], preferred_element_type=jnp.float32)
            + jnp.dot(x_ref[...], wt_ref[...], preferred_element_type=jnp.float32)
            + b_ref[...]
        )

    return pl.pallas_call(
        body,
        grid=(N_NODES // BLK,),
        in_specs=[
            pl.BlockSpec((BLK, D), lambda i: (i, 0)),
            pl.BlockSpec((BLK, D), lambda i: (i, 0)),
            pl.BlockSpec((BLK, D), lambda i: (i, 0)),
            pl.BlockSpec((D, D), lambda i: (0, 0)),
            pl.BlockSpec((D, D), lambda i: (0, 0)),
            pl.BlockSpec((1, D), lambda i: (0, 0)),
        ],
        out_specs=pl.BlockSpec((BLK, D), lambda i: (i, 0)),
        out_shape=jax.ShapeDtypeStruct((N_NODES, D), jnp.float32),
    )(p0, p1, x, wr_t, wt_t, b)


def kernel(x, edge_index, W_rel, b_rel, W_root):
    src = edge_index[0].astype(jnp.int32)
    dst = edge_index[1].astype(jnp.int32)
    zeros = jnp.zeros((N_PAD, D), jnp.float32)
    partials = _sc_segment_sum(x, src, dst, zeros)
    return _tc_combine(partials[0, :N_NODES], partials[1, :N_NODES], x,
                       W_rel.T, W_root.T, b_rel.reshape(1, D))
